# Initial kernel scaffold; baseline (speedup 1.0000x reference)
import jax
import jax.numpy as jnp
from jax.experimental import pallas as pl


def kernel(x, edge_index, in_W, in_b, l0_qparams, l0_lin_W, l0_lin_b, l0_qp_W, l0_qp_b, l1_qparams, l1_lin_W, l1_lin_b, l1_qp_W, l1_qp_b, out_W, out_b):
    raise NotImplementedError("write your pallas kernel here")


# speedup vs baseline: 6.2242x; 6.2242x over previous
import functools

import numpy as np
import jax
import jax.numpy as jnp
from jax import lax
from jax.experimental import pallas as pl
from jax.experimental.pallas import tpu as pltpu
from jax.experimental.pallas import tpu_sc as plsc

_N = 10000
_D = 128
_NW = 32
_CH = 128
_ACC_PER_TILE = 640
_ACC_ROWS = 16 * _ACC_PER_TILE
_ROW_BLK = 2000


def _xc_from_h(h, lWt, lb, qWt, c1, c2):
    a = jnp.tanh(h[:, 0:4]) * np.float32(np.pi / 2.0)
    z = jnp.cos(a) * c1[:, 0:4] - jnp.sin(a) * c2[:, 0:4]
    q0 = z[:, 0:1]
    q1 = q0 * z[:, 1:2]
    q2 = q1 * z[:, 2:3]
    q3 = q2 * z[:, 3:4]
    xq = q0 * qWt[0:1, :] + q1 * qWt[1:2, :] + q2 * qWt[2:3, :] + q3 * qWt[3:4, :]
    return jnp.dot(h, lWt, preferred_element_type=jnp.float32) + lb + xq


def _tc_first_body(x_ref, inWt_ref, inb_ref, lWt_ref, lb_ref, qWt_ref, c1_ref,
                   c2_ref, h_ref, xc_ref):
    h = jax.nn.relu(
        jnp.dot(x_ref[...], inWt_ref[...], preferred_element_type=jnp.float32)
        + inb_ref[...])
    h_ref[...] = h
    xc_ref[...] = _xc_from_h(h, lWt_ref[...], lb_ref[...], qWt_ref[...],
                             c1_ref[...], c2_ref[...])


def _tc_mid_body(hp_ref, xcp_ref, acc0_ref, acc1_ref, lWt_ref, lb_ref, qWt_ref,
                 c1_ref, c2_ref, h_ref, xc_ref):
    agg = acc0_ref[...] + acc1_ref[...] + xcp_ref[...]
    h = hp_ref[...] + jax.nn.relu(agg)
    h_ref[...] = h
    xc_ref[...] = _xc_from_h(h, lWt_ref[...], lb_ref[...], qWt_ref[...],
                             c1_ref[...], c2_ref[...])


def _tc_last_body(hp_ref, xcp_ref, acc0_ref, acc1_ref, outWt_ref, outb_ref,
                  out_ref):
    agg = acc0_ref[...] + acc1_ref[...] + xcp_ref[...]
    h = hp_ref[...] + jax.nn.relu(agg)
    out_ref[...] = (
        jnp.dot(h, outWt_ref[...], preferred_element_type=jnp.float32)
        + outb_ref[...])


_row_spec = pl.BlockSpec((_ROW_BLK, _D), lambda i: (i, 0))
_mat_spec = pl.BlockSpec((_D, _D), lambda i: (0, 0))
_vec_spec = pl.BlockSpec((1, _D), lambda i: (0, 0))
_qw_spec = pl.BlockSpec((8, _D), lambda i: (0, 0))
_GRID = (_N // _ROW_BLK,)

_hx_shape = (jax.ShapeDtypeStruct((_N, _D), jnp.float32),
             jax.ShapeDtypeStruct((_N, _D), jnp.float32))

_tc_first = pl.pallas_call(
    _tc_first_body, grid=_GRID,
    in_specs=[_row_spec, _mat_spec, _vec_spec, _mat_spec, _vec_spec, _qw_spec,
              _vec_spec, _vec_spec],
    out_specs=(_row_spec, _row_spec),
    out_shape=_hx_shape)

_tc_mid = pl.pallas_call(
    _tc_mid_body, grid=_GRID,
    in_specs=[_row_spec, _row_spec, _row_spec, _row_spec, _mat_spec, _vec_spec,
              _qw_spec, _vec_spec, _vec_spec],
    out_specs=(_row_spec, _row_spec),
    out_shape=_hx_shape)

_tc_last = pl.pallas_call(
    _tc_last_body, grid=_GRID,
    in_specs=[_row_spec, _row_spec, _row_spec, _row_spec, _mat_spec, _vec_spec],
    out_specs=_row_spec,
    out_shape=jax.ShapeDtypeStruct((_N, _D), jnp.float32))


def _sc_scatter_body(xc_hbm, srcb_hbm, dstb_hbm, zblk_hbm, out_hbm,
                     src_v, dst_v, rows_v, acc_sh, sem):
    c = lax.axis_index("c")
    s = lax.axis_index("s")
    g = s * 2 + c

    pltpu.sync_copy(zblk_hbm, acc_sh.at[pl.ds(s * _ACC_PER_TILE, _ACC_PER_TILE)])
    plsc.subcore_barrier()

    pltpu.sync_copy(srcb_hbm.at[g], src_v)
    pltpu.sync_copy(dstb_hbm.at[g], dst_v)

    nchunks = src_v.shape[0]

    def chunk(j, carry):
        pltpu.async_copy(xc_hbm.at[src_v.at[j]], rows_v, sem).wait()
        pltpu.sync_copy(rows_v, acc_sh.at[dst_v.at[j]], add=True)
        return carry

    lax.fori_loop(0, nchunks, chunk, 0)
    plsc.subcore_barrier()

    pltpu.sync_copy(acc_sh.at[pl.ds(s * _ACC_PER_TILE, _ACC_PER_TILE)],
                    out_hbm.at[c].at[pl.ds(s * _ACC_PER_TILE, _ACC_PER_TILE)])


def _make_sc_scatter(nchunks):
    mesh = plsc.VectorSubcoreMesh(core_axis_name="c", subcore_axis_name="s")
    return pl.kernel(
        _sc_scatter_body,
        out_type=jax.ShapeDtypeStruct((2, _ACC_ROWS, _D), jnp.float32),
        mesh=mesh,
        scratch_types=[
            pltpu.VMEM((nchunks, _CH), jnp.int32),
            pltpu.VMEM((nchunks, _CH), jnp.int32),
            pltpu.VMEM((_CH, _D), jnp.float32),
            pltpu.VMEM_SHARED((_ACC_ROWS, _D), jnp.float32),
            pltpu.SemaphoreType.DMA,
        ])


def kernel(x, edge_index, in_W, in_b, l0_qparams, l0_lin_W, l0_lin_b, l0_qp_W,
           l0_qp_b, l1_qparams, l1_lin_W, l1_lin_b, l1_qp_W, l1_qp_b, out_W,
           out_b):
    def prep_layer(qparams, lin_W, lin_b, qp_W, qp_b):
        lWt = lin_W.T
        lb = (lin_b + qp_b).reshape(1, _D)
        qWt = jnp.zeros((8, _D), jnp.float32).at[0:4].set(qp_W.T)
        phi, th = qparams[:, 0], qparams[:, 1]
        c1 = jnp.zeros((1, _D), jnp.float32).at[0, 0:4].set(jnp.cos(th))
        c2 = jnp.zeros((1, _D), jnp.float32).at[0, 0:4].set(
            jnp.sin(th) * jnp.cos(phi))
        return lWt, lb, qWt, c1, c2

    l0 = prep_layer(l0_qparams, l0_lin_W, l0_lin_b, l0_qp_W, l0_qp_b)
    l1 = prep_layer(l1_qparams, l1_lin_W, l1_lin_b, l1_qp_W, l1_qp_b)
    inWt = in_W.T
    inb = in_b.reshape(1, _D)
    outWt = out_W.T
    outb = out_b.reshape(1, _D)

    src = edge_index[0]
    dst = edge_index[1]
    n_edges = src.shape[0]
    nchunks = -(-n_edges // (_NW * _CH))
    pad = _NW * nchunks * _CH - n_edges
    src_p = jnp.concatenate([src, jnp.zeros((pad,), jnp.int32)])
    dst_p = jnp.concatenate([dst, jnp.full((pad,), _N, jnp.int32)])
    srcb = src_p.reshape(_NW, nchunks, _CH)
    dstb = dst_p.reshape(_NW, nchunks, _CH)
    zblk = jnp.zeros((_ACC_PER_TILE, _D), jnp.float32)

    sc_scatter = _make_sc_scatter(nchunks)

    h, xc = _tc_first(x, inWt, inb, *l0)
    acc = sc_scatter(xc, srcb, dstb, zblk)
    h, xc = _tc_mid(h, xc, acc[0, :_N], acc[1, :_N], *l1)
    acc = sc_scatter(xc, srcb, dstb, zblk)
    return _tc_last(h, xc, acc[0, :_N], acc[1, :_N], outWt, outb)


# speedup vs baseline: 7.8877x; 1.2673x over previous
import functools

import numpy as np
import jax
import jax.numpy as jnp
from jax import lax
from jax.experimental import pallas as pl
from jax.experimental.pallas import tpu as pltpu
from jax.experimental.pallas import tpu_sc as plsc

_N = 10000
_D = 128
_NW = 32
_CH = 128
_ACC_PER_TILE = 632
_ACC_ROWS = 10008
_ROW_BLK = 2000


def _xc_from_h(h, lWt, lb, qWt, c1, c2):
    a = jnp.tanh(h) * np.float32(np.pi / 2.0)
    z = jnp.cos(a) * c1 - jnp.sin(a) * c2
    lane = lax.broadcasted_iota(jnp.int32, z.shape, 1)
    one = jnp.float32(1.0)
    t1 = z * jnp.where(lane == 0, one, pltpu.roll(z, 1, 1))
    t2 = t1 * jnp.where(lane < 2, one, pltpu.roll(t1, 2, 1))
    return (jnp.dot(h, lWt, preferred_element_type=jnp.float32) + lb
            + jnp.dot(t2, qWt, preferred_element_type=jnp.float32))


def _tc_first_body(x_ref, inWt_ref, inb_ref, lWt_ref, lb_ref, qWt_ref, c1_ref,
                   c2_ref, h_ref, xc_ref):
    h = jax.nn.relu(
        jnp.dot(x_ref[...], inWt_ref[...], preferred_element_type=jnp.float32)
        + inb_ref[...])
    h_ref[...] = h
    xc_ref[...] = _xc_from_h(h, lWt_ref[...], lb_ref[...], qWt_ref[...],
                             c1_ref[...], c2_ref[...])


def _tc_mid_body(hp_ref, xcp_ref, acc0_ref, acc1_ref, lWt_ref, lb_ref, qWt_ref,
                 c1_ref, c2_ref, h_ref, xc_ref):
    agg = acc0_ref[...] + acc1_ref[...] + xcp_ref[...]
    h = hp_ref[...] + jax.nn.relu(agg)
    h_ref[...] = h
    xc_ref[...] = _xc_from_h(h, lWt_ref[...], lb_ref[...], qWt_ref[...],
                             c1_ref[...], c2_ref[...])


def _tc_last_body(hp_ref, xcp_ref, acc0_ref, acc1_ref, outWt_ref, outb_ref,
                  out_ref):
    agg = acc0_ref[...] + acc1_ref[...] + xcp_ref[...]
    h = hp_ref[...] + jax.nn.relu(agg)
    out_ref[...] = (
        jnp.dot(h, outWt_ref[...], preferred_element_type=jnp.float32)
        + outb_ref[...])


_row_spec = pl.BlockSpec((_ROW_BLK, _D), lambda i: (i, 0))
_mat_spec = pl.BlockSpec((_D, _D), lambda i: (0, 0))
_vec_spec = pl.BlockSpec((1, _D), lambda i: (0, 0))
_qw_spec = pl.BlockSpec((_D, _D), lambda i: (0, 0))
_GRID = (_N // _ROW_BLK,)

_hx_shape = (jax.ShapeDtypeStruct((_N, _D), jnp.float32),
             jax.ShapeDtypeStruct((_N, _D), jnp.float32))

_tc_first = pl.pallas_call(
    _tc_first_body, grid=_GRID,
    in_specs=[_row_spec, _mat_spec, _vec_spec, _mat_spec, _vec_spec, _qw_spec,
              _vec_spec, _vec_spec],
    out_specs=(_row_spec, _row_spec),
    out_shape=_hx_shape)

_tc_mid = pl.pallas_call(
    _tc_mid_body, grid=_GRID,
    in_specs=[_row_spec, _row_spec, _row_spec, _row_spec, _mat_spec, _vec_spec,
              _qw_spec, _vec_spec, _vec_spec],
    out_specs=(_row_spec, _row_spec),
    out_shape=_hx_shape)

_tc_last = pl.pallas_call(
    _tc_last_body, grid=_GRID,
    in_specs=[_row_spec, _row_spec, _row_spec, _row_spec, _mat_spec, _vec_spec],
    out_specs=_row_spec,
    out_shape=jax.ShapeDtypeStruct((_N, _D), jnp.float32))


_NBUF = 3
_NIDX = 6


def _sc_scatter_body(xc_hbm, eib_hbm, zblk_hbm, out_hbm,
                     idx_v, rows_v, acc_sh, gsems, ssems, isems):
    c = lax.axis_index("c")
    s = lax.axis_index("s")
    g = s * 2 + c
    nchunks = eib_hbm.shape[1]

    @pl.when(s < 15)
    def _():
        pltpu.sync_copy(zblk_hbm,
                        acc_sh.at[pl.ds(s * _ACC_PER_TILE, _ACC_PER_TILE)])

    @pl.when(s == 15)
    def _():
        pltpu.sync_copy(zblk_hbm.at[pl.ds(0, _ACC_ROWS - 15 * _ACC_PER_TILE)],
                        acc_sh.at[pl.ds(15 * _ACC_PER_TILE,
                                        _ACC_ROWS - 15 * _ACC_PER_TILE)])

    plsc.subcore_barrier()

    def start_idx_load(j, bi):
        pltpu.async_copy(eib_hbm.at[g, j], idx_v.at[bi], isems.at[bi])

    def start_gather(bi, b):
        pltpu.async_copy(xc_hbm.at[idx_v.at[bi, 0]], rows_v.at[b],
                         gsems.at[b])

    def start_scatter(bi, b):
        pltpu.async_copy(rows_v.at[b], acc_sh.at[idx_v.at[bi, 1]], ssems.at[b],
                         add=True)

    def drain_rows(sems, b):
        pltpu.make_async_copy(xc_hbm.at[pl.ds(0, _CH)], rows_v.at[b],
                              sems.at[b]).wait()

    def drain_idx(bi):
        pltpu.make_async_copy(eib_hbm.at[0, 0], idx_v.at[bi],
                              isems.at[bi]).wait()

    for j0 in range(_NBUF):
        start_idx_load(j0, j0)

    def step(i, bi):
        b_g = bi % _NBUF
        b_s = (bi + 1) % _NBUF

        @pl.when(jnp.logical_and(i >= _NBUF, i < nchunks + _NBUF))
        def _():
            drain_rows(ssems, b_g)

        li = i + _NBUF

        @pl.when(li < nchunks)
        def _():
            start_idx_load(li, (bi + _NBUF) % _NIDX)

        @pl.when(i < nchunks)
        def _():
            drain_idx(bi)
            start_gather(bi, b_g)

        sj = i - (_NBUF - 1)

        @pl.when(jnp.logical_and(sj >= 0, sj < nchunks))
        def _():
            drain_rows(gsems, b_s)
            start_scatter((bi + _NIDX - (_NBUF - 1)) % _NIDX, b_s)

    n_outer = (nchunks + 2 * _NBUF + _NIDX - 1) // _NIDX

    def group(t, carry):
        for bi in range(_NIDX):
            step(t * _NIDX + bi, bi)
        return carry

    lax.fori_loop(0, n_outer, group, 0)
    plsc.subcore_barrier()

    @pl.when(s < 15)
    def _():
        pltpu.sync_copy(acc_sh.at[pl.ds(s * _ACC_PER_TILE, _ACC_PER_TILE)],
                        out_hbm.at[c].at[pl.ds(s * _ACC_PER_TILE,
                                               _ACC_PER_TILE)])

    @pl.when(s == 15)
    def _():
        pltpu.sync_copy(acc_sh.at[pl.ds(15 * _ACC_PER_TILE,
                                        _ACC_ROWS - 15 * _ACC_PER_TILE)],
                        out_hbm.at[c].at[pl.ds(15 * _ACC_PER_TILE,
                                               _ACC_ROWS - 15 * _ACC_PER_TILE)])


def _make_sc_scatter(nchunks):
    mesh = plsc.VectorSubcoreMesh(core_axis_name="c", subcore_axis_name="s")
    return pl.kernel(
        _sc_scatter_body,
        out_type=jax.ShapeDtypeStruct((2, _ACC_ROWS, _D), jnp.float32),
        mesh=mesh,
        scratch_types=[
            pltpu.VMEM((_NIDX, 2, _CH), jnp.int32),
            pltpu.VMEM((_NBUF, _CH, _D), jnp.float32),
            pltpu.VMEM_SHARED((_ACC_ROWS, _D), jnp.float32),
            pltpu.SemaphoreType.DMA((_NBUF,)),
            pltpu.SemaphoreType.DMA((_NBUF,)),
            pltpu.SemaphoreType.DMA((_NIDX,)),
        ])


def kernel(x, edge_index, in_W, in_b, l0_qparams, l0_lin_W, l0_lin_b, l0_qp_W,
           l0_qp_b, l1_qparams, l1_lin_W, l1_lin_b, l1_qp_W, l1_qp_b, out_W,
           out_b):
    def prep_layer(qparams, lin_W, lin_b, qp_W, qp_b):
        lWt = lin_W.T
        lb = (lin_b + qp_b).reshape(1, _D)
        qWt = jnp.zeros((_D, _D), jnp.float32).at[0:4].set(qp_W.T)
        phi, th = qparams[:, 0], qparams[:, 1]
        c1 = jnp.zeros((1, _D), jnp.float32).at[0, 0:4].set(jnp.cos(th))
        c2 = jnp.zeros((1, _D), jnp.float32).at[0, 0:4].set(
            jnp.sin(th) * jnp.cos(phi))
        return lWt, lb, qWt, c1, c2

    l0 = prep_layer(l0_qparams, l0_lin_W, l0_lin_b, l0_qp_W, l0_qp_b)
    l1 = prep_layer(l1_qparams, l1_lin_W, l1_lin_b, l1_qp_W, l1_qp_b)
    inWt = in_W.T
    inb = in_b.reshape(1, _D)
    outWt = out_W.T
    outb = out_b.reshape(1, _D)

    src = edge_index[0]
    dst = edge_index[1]
    n_edges = src.shape[0]
    nchunks = -(-n_edges // (_NW * _CH))
    pad = _NW * nchunks * _CH - n_edges
    src_p = jnp.concatenate([src, jnp.zeros((pad,), jnp.int32)])
    dst_p = jnp.concatenate([dst, jnp.full((pad,), _N, jnp.int32)])
    eib = jnp.stack([src_p.reshape(_NW, nchunks, _CH),
                     dst_p.reshape(_NW, nchunks, _CH)], axis=2)
    zblk = jnp.zeros((_ACC_PER_TILE, _D), jnp.float32)

    sc_scatter = _make_sc_scatter(nchunks)

    h, xc = _tc_first(x, inWt, inb, *l0)
    acc = sc_scatter(xc, eib, zblk)
    h, xc = _tc_mid(h, xc, acc[0, :_N], acc[1, :_N], *l1)
    acc = sc_scatter(xc, eib, zblk)
    return _tc_last(h, xc, acc[0, :_N], acc[1, :_N], outWt, outb)


# speedup vs baseline: 12.0992x; 1.5339x over previous
import functools

import numpy as np
import jax
import jax.numpy as jnp
from jax import lax
from jax.experimental import pallas as pl
from jax.experimental.pallas import tpu as pltpu
from jax.experimental.pallas import tpu_sc as plsc

_N = 10000
_D = 128
_NW = 32
_CH = 128
_ACC_PER_TILE = 632
_ACC_ROWS = 10008
_ROW_BLK = 2000


def _xc_from_h(h, lWt, lb, qWt, c1, c2):
    a = jnp.tanh(h) * np.float32(np.pi / 2.0)
    z = jnp.cos(a) * c1 - jnp.sin(a) * c2
    lane = lax.broadcasted_iota(jnp.int32, z.shape, 1)
    one = jnp.float32(1.0)
    t1 = z * jnp.where(lane == 0, one, pltpu.roll(z, 1, 1))
    t2 = t1 * jnp.where(lane < 2, one, pltpu.roll(t1, 2, 1))
    return (jnp.dot(h, lWt, preferred_element_type=jnp.float32) + lb
            + jnp.dot(t2, qWt, preferred_element_type=jnp.float32))


def _tc_first_body(x_ref, inWt_ref, inb_ref, lWt_ref, lb_ref, qWt_ref, c1_ref,
                   c2_ref, h_ref, xc_ref):
    h = jax.nn.relu(
        jnp.dot(x_ref[...], inWt_ref[...], preferred_element_type=jnp.float32)
        + inb_ref[...])
    h_ref[...] = h
    xc_ref[...] = _xc_from_h(h, lWt_ref[...], lb_ref[...], qWt_ref[...],
                             c1_ref[...], c2_ref[...])


def _tc_mid_body(hp_ref, xcp_ref, acc0_ref, acc1_ref, lWt_ref, lb_ref, qWt_ref,
                 c1_ref, c2_ref, h_ref, xc_ref):
    agg = acc0_ref[...] + acc1_ref[...] + xcp_ref[...]
    h = hp_ref[...] + jax.nn.relu(agg)
    h_ref[...] = h
    xc_ref[...] = _xc_from_h(h, lWt_ref[...], lb_ref[...], qWt_ref[...],
                             c1_ref[...], c2_ref[...])


def _tc_last_body(hp_ref, xcp_ref, acc0_ref, acc1_ref, outWt_ref, outb_ref,
                  out_ref):
    agg = acc0_ref[...] + acc1_ref[...] + xcp_ref[...]
    h = hp_ref[...] + jax.nn.relu(agg)
    out_ref[...] = (
        jnp.dot(h, outWt_ref[...], preferred_element_type=jnp.float32)
        + outb_ref[...])


_row_spec = pl.BlockSpec((_ROW_BLK, _D), lambda i: (i, 0))
_mat_spec = pl.BlockSpec((_D, _D), lambda i: (0, 0))
_vec_spec = pl.BlockSpec((1, _D), lambda i: (0, 0))
_qw_spec = pl.BlockSpec((_D, _D), lambda i: (0, 0))
_GRID = (_N // _ROW_BLK,)

_hx_shape = (jax.ShapeDtypeStruct((_N, _D), jnp.float32),
             jax.ShapeDtypeStruct((_N, _D), jnp.float32))

_tc_first = pl.pallas_call(
    _tc_first_body, grid=_GRID,
    in_specs=[_row_spec, _mat_spec, _vec_spec, _mat_spec, _vec_spec, _qw_spec,
              _vec_spec, _vec_spec],
    out_specs=(_row_spec, _row_spec),
    out_shape=_hx_shape)

_tc_mid = pl.pallas_call(
    _tc_mid_body, grid=_GRID,
    in_specs=[_row_spec, _row_spec, _row_spec, _row_spec, _mat_spec, _vec_spec,
              _qw_spec, _vec_spec, _vec_spec],
    out_specs=(_row_spec, _row_spec),
    out_shape=_hx_shape)

_tc_last = pl.pallas_call(
    _tc_last_body, grid=_GRID,
    in_specs=[_row_spec, _row_spec, _row_spec, _row_spec, _mat_spec, _vec_spec],
    out_specs=_row_spec,
    out_shape=jax.ShapeDtypeStruct((_N, _D), jnp.float32))


_NBUF = 3
_NIDX = 6
_CORE0_FRAC = 0.72


def _sc_scatter_body(n0, n1, xc_hbm, eib_hbm, zblk_hbm, out_hbm,
                     idx_v, rows_v, acc_sh, gsems, ssems, isems):
    c = lax.axis_index("c")
    s = lax.axis_index("s")
    g = s * 2 + c
    nchunks = jnp.where(c == 0, n0, n1)

    @pl.when(s < 15)
    def _():
        pltpu.sync_copy(zblk_hbm,
                        acc_sh.at[pl.ds(s * _ACC_PER_TILE, _ACC_PER_TILE)])

    @pl.when(s == 15)
    def _():
        pltpu.sync_copy(zblk_hbm.at[pl.ds(0, _ACC_ROWS - 15 * _ACC_PER_TILE)],
                        acc_sh.at[pl.ds(15 * _ACC_PER_TILE,
                                        _ACC_ROWS - 15 * _ACC_PER_TILE)])

    plsc.subcore_barrier()

    def start_idx_load(j, bi):
        pltpu.async_copy(eib_hbm.at[g, j], idx_v.at[bi], isems.at[bi])

    def start_gather(bi, b):
        pltpu.async_copy(xc_hbm.at[idx_v.at[bi, 0]], rows_v.at[b],
                         gsems.at[b])

    def start_scatter(bi, b):
        pltpu.async_copy(rows_v.at[b], acc_sh.at[idx_v.at[bi, 1]], ssems.at[b],
                         add=True)

    def drain_rows(sems, b):
        pltpu.make_async_copy(xc_hbm.at[pl.ds(0, _CH)], rows_v.at[b],
                              sems.at[b]).wait()

    def drain_idx(bi):
        pltpu.make_async_copy(eib_hbm.at[0, 0], idx_v.at[bi],
                              isems.at[bi]).wait()

    for j0 in range(_NBUF):
        @pl.when(j0 < nchunks)
        def _():
            start_idx_load(j0, j0)

    def step(i, bi):
        b_g = bi % _NBUF
        b_s = (bi + 1) % _NBUF

        @pl.when(jnp.logical_and(i >= _NBUF, i < nchunks + _NBUF))
        def _():
            drain_rows(ssems, b_g)

        li = i + _NBUF

        @pl.when(li < nchunks)
        def _():
            start_idx_load(li, (bi + _NBUF) % _NIDX)

        @pl.when(i < nchunks)
        def _():
            drain_idx(bi)
            start_gather(bi, b_g)

        sj = i - (_NBUF - 1)

        @pl.when(jnp.logical_and(sj >= 0, sj < nchunks))
        def _():
            drain_rows(gsems, b_s)
            start_scatter((bi + _NIDX - (_NBUF - 1)) % _NIDX, b_s)

    n_outer = (max(n0, n1) + 2 * _NBUF + _NIDX - 1) // _NIDX

    def group(t, carry):
        for bi in range(_NIDX):
            step(t * _NIDX + bi, bi)
        return carry

    lax.fori_loop(0, n_outer, group, 0)
    plsc.subcore_barrier()

    @pl.when(s < 15)
    def _():
        pltpu.sync_copy(acc_sh.at[pl.ds(s * _ACC_PER_TILE, _ACC_PER_TILE)],
                        out_hbm.at[c].at[pl.ds(s * _ACC_PER_TILE,
                                               _ACC_PER_TILE)])

    @pl.when(s == 15)
    def _():
        pltpu.sync_copy(acc_sh.at[pl.ds(15 * _ACC_PER_TILE,
                                        _ACC_ROWS - 15 * _ACC_PER_TILE)],
                        out_hbm.at[c].at[pl.ds(15 * _ACC_PER_TILE,
                                               _ACC_ROWS - 15 * _ACC_PER_TILE)])


def _make_sc_scatter(n0, n1):
    mesh = plsc.VectorSubcoreMesh(core_axis_name="c", subcore_axis_name="s")
    return pl.kernel(
        functools.partial(_sc_scatter_body, n0, n1),
        out_type=jax.ShapeDtypeStruct((2, _ACC_ROWS, _D), jnp.float32),
        mesh=mesh,
        scratch_types=[
            pltpu.VMEM((_NIDX, 2, _CH), jnp.int32),
            pltpu.VMEM((_NBUF, _CH, _D), jnp.float32),
            pltpu.VMEM_SHARED((_ACC_ROWS, _D), jnp.float32),
            pltpu.SemaphoreType.DMA((_NBUF,)),
            pltpu.SemaphoreType.DMA((_NBUF,)),
            pltpu.SemaphoreType.DMA((_NIDX,)),
        ])


def kernel(x, edge_index, in_W, in_b, l0_qparams, l0_lin_W, l0_lin_b, l0_qp_W,
           l0_qp_b, l1_qparams, l1_lin_W, l1_lin_b, l1_qp_W, l1_qp_b, out_W,
           out_b):
    def prep_layer(qparams, lin_W, lin_b, qp_W, qp_b):
        lWt = lin_W.T
        lb = (lin_b + qp_b).reshape(1, _D)
        qWt = jnp.zeros((_D, _D), jnp.float32).at[0:4].set(qp_W.T)
        phi, th = qparams[:, 0], qparams[:, 1]
        c1 = jnp.zeros((1, _D), jnp.float32).at[0, 0:4].set(jnp.cos(th))
        c2 = jnp.zeros((1, _D), jnp.float32).at[0, 0:4].set(
            jnp.sin(th) * jnp.cos(phi))
        return lWt, lb, qWt, c1, c2

    l0 = prep_layer(l0_qparams, l0_lin_W, l0_lin_b, l0_qp_W, l0_qp_b)
    l1 = prep_layer(l1_qparams, l1_lin_W, l1_lin_b, l1_qp_W, l1_qp_b)
    inWt = in_W.T
    inb = in_b.reshape(1, _D)
    outWt = out_W.T
    outb = out_b.reshape(1, _D)

    src = edge_index[0]
    dst = edge_index[1]
    n_edges = src.shape[0]
    n_pair = -(-n_edges // (16 * _CH))
    n0 = max(1, min(n_pair - 1, round(n_pair * _CORE0_FRAC)))
    n1 = n_pair - n0
    pad = 16 * n_pair * _CH - n_edges
    src_p = jnp.concatenate([src, jnp.zeros((pad,), jnp.int32)])
    dst_p = jnp.concatenate([dst, jnp.full((pad,), _N, jnp.int32)])
    ei = jnp.stack([src_p.reshape(-1, _CH), dst_p.reshape(-1, _CH)], axis=1)
    nmax = max(n0, n1)
    eia = ei[:16 * n0].reshape(16, n0, 2, _CH)
    eib1 = ei[16 * n0:].reshape(16, n1, 2, _CH)
    eia = jnp.pad(eia, ((0, 0), (0, nmax - n0), (0, 0), (0, 0)))
    eib1 = jnp.pad(eib1, ((0, 0), (0, nmax - n1), (0, 0), (0, 0)))
    eib = jnp.stack([eia, eib1], axis=1).reshape(_NW, nmax, 2, _CH)
    zblk = jnp.zeros((_ACC_PER_TILE, _D), jnp.float32)

    sc_scatter = _make_sc_scatter(n0, n1)

    h, xc = _tc_first(x, inWt, inb, *l0)
    acc = sc_scatter(xc, eib, zblk)
    h, xc = _tc_mid(h, xc, acc[0, :_N], acc[1, :_N], *l1)
    acc = sc_scatter(xc, eib, zblk)
    return _tc_last(h, xc, acc[0, :_N], acc[1, :_N], outWt, outb)


# speedup vs baseline: 12.9475x; 1.0701x over previous
import functools

import numpy as np
import jax
import jax.numpy as jnp
from jax import lax
from jax.experimental import pallas as pl
from jax.experimental.pallas import tpu as pltpu
from jax.experimental.pallas import tpu_sc as plsc

_N = 10000
_D = 128
_NW = 32
_CH = 128
_ACC_PER_TILE = 632
_ACC_ROWS = 10008
_ROW_BLK = 2000


def _xc_from_h(h, lWt, lb, qWt, c1, c2, sel):
    h8t = lax.dot_general(sel, h, (((1,), (1,)), ((), ())),
                          preferred_element_type=jnp.float32)
    a = jnp.tanh(h8t) * np.float32(np.pi / 2.0)
    z = jnp.cos(a) * c1[:, 0:1] - jnp.sin(a) * c2[:, 0:1]
    sub = lax.broadcasted_iota(jnp.int32, z.shape, 0)
    one = jnp.float32(1.0)
    t1 = z * jnp.where(sub == 0, one, pltpu.roll(z, 1, 0))
    t2 = t1 * jnp.where(sub < 2, one, pltpu.roll(t1, 2, 0))
    xq = lax.dot_general(t2, qWt, (((0,), (0,)), ((), ())),
                         preferred_element_type=jnp.float32)
    return jnp.dot(h, lWt, preferred_element_type=jnp.float32) + lb + xq


def _tc_first_body(x_ref, inWt_ref, inb_ref, lWt_ref, lb_ref, qWt_ref, c1_ref,
                   c2_ref, sel_ref, h_ref, xc_ref):
    h = jax.nn.relu(
        jnp.dot(x_ref[...], inWt_ref[...], preferred_element_type=jnp.float32)
        + inb_ref[...])
    h_ref[...] = h
    xc_ref[...] = _xc_from_h(h, lWt_ref[...], lb_ref[...], qWt_ref[...],
                             c1_ref[...], c2_ref[...], sel_ref[...])


def _tc_mid_body(hp_ref, xcp_ref, acc0_ref, acc1_ref, lWt_ref, lb_ref, qWt_ref,
                 c1_ref, c2_ref, sel_ref, h_ref, xc_ref):
    agg = acc0_ref[...] + acc1_ref[...] + xcp_ref[...]
    h = hp_ref[...] + jax.nn.relu(agg)
    h_ref[...] = h
    xc_ref[...] = _xc_from_h(h, lWt_ref[...], lb_ref[...], qWt_ref[...],
                             c1_ref[...], c2_ref[...], sel_ref[...])


def _tc_last_body(hp_ref, xcp_ref, acc0_ref, acc1_ref, outWt_ref, outb_ref,
                  out_ref):
    agg = acc0_ref[...] + acc1_ref[...] + xcp_ref[...]
    h = hp_ref[...] + jax.nn.relu(agg)
    out_ref[...] = (
        jnp.dot(h, outWt_ref[...], preferred_element_type=jnp.float32)
        + outb_ref[...])


_row_spec = pl.BlockSpec((_ROW_BLK, _D), lambda i: (i, 0))
_mat_spec = pl.BlockSpec((_D, _D), lambda i: (0, 0))
_vec_spec = pl.BlockSpec((1, _D), lambda i: (0, 0))
_qw_spec = pl.BlockSpec((8, _D), lambda i: (0, 0))
_GRID = (_N // _ROW_BLK,)

_hx_shape = (jax.ShapeDtypeStruct((_N, _D), jnp.float32),
             jax.ShapeDtypeStruct((_N, _D), jnp.float32))

_tc_first = pl.pallas_call(
    _tc_first_body, grid=_GRID,
    in_specs=[_row_spec, _mat_spec, _vec_spec, _mat_spec, _vec_spec, _qw_spec,
              _qw_spec, _qw_spec, _qw_spec],
    out_specs=(_row_spec, _row_spec),
    out_shape=_hx_shape)

_tc_mid = pl.pallas_call(
    _tc_mid_body, grid=_GRID,
    in_specs=[_row_spec, _row_spec, _row_spec, _row_spec, _mat_spec, _vec_spec,
              _qw_spec, _qw_spec, _qw_spec, _qw_spec],
    out_specs=(_row_spec, _row_spec),
    out_shape=_hx_shape)

_tc_last = pl.pallas_call(
    _tc_last_body, grid=_GRID,
    in_specs=[_row_spec, _row_spec, _row_spec, _row_spec, _mat_spec, _vec_spec],
    out_specs=_row_spec,
    out_shape=jax.ShapeDtypeStruct((_N, _D), jnp.float32))


_NBUF = 3
_NIDX = 6
_CORE0_FRAC = 0.70


def _sc_scatter_body(n0, n1, xc_hbm, eib_hbm, zblk_hbm, out_hbm,
                     idx_v, rows_v, acc_sh, gsems, ssems, isems):
    c = lax.axis_index("c")
    s = lax.axis_index("s")
    g = s * 2 + c
    nchunks = jnp.where(c == 0, n0, n1)

    @pl.when(s < 15)
    def _():
        pltpu.sync_copy(zblk_hbm,
                        acc_sh.at[pl.ds(s * _ACC_PER_TILE, _ACC_PER_TILE)])

    @pl.when(s == 15)
    def _():
        pltpu.sync_copy(zblk_hbm.at[pl.ds(0, _ACC_ROWS - 15 * _ACC_PER_TILE)],
                        acc_sh.at[pl.ds(15 * _ACC_PER_TILE,
                                        _ACC_ROWS - 15 * _ACC_PER_TILE)])

    plsc.subcore_barrier()

    def start_idx_load(j, bi):
        pltpu.async_copy(eib_hbm.at[g, j], idx_v.at[bi], isems.at[bi])

    def start_gather(bi, b):
        pltpu.async_copy(xc_hbm.at[idx_v.at[bi, 0]], rows_v.at[b],
                         gsems.at[b])

    def start_scatter(bi, b):
        pltpu.async_copy(rows_v.at[b], acc_sh.at[idx_v.at[bi, 1]], ssems.at[b],
                         add=True)

    def drain_rows(sems, b):
        pltpu.make_async_copy(xc_hbm.at[pl.ds(0, _CH)], rows_v.at[b],
                              sems.at[b]).wait()

    def drain_idx(bi):
        pltpu.make_async_copy(eib_hbm.at[0, 0], idx_v.at[bi],
                              isems.at[bi]).wait()

    for j0 in range(_NBUF):
        @pl.when(j0 < nchunks)
        def _():
            start_idx_load(j0, j0)

    def step(i, bi):
        b_g = bi % _NBUF
        b_s = (bi + 1) % _NBUF

        @pl.when(jnp.logical_and(i >= _NBUF, i < nchunks + _NBUF))
        def _():
            drain_rows(ssems, b_g)

        li = i + _NBUF

        @pl.when(li < nchunks)
        def _():
            start_idx_load(li, (bi + _NBUF) % _NIDX)

        @pl.when(i < nchunks)
        def _():
            drain_idx(bi)
            start_gather(bi, b_g)

        sj = i - (_NBUF - 1)

        @pl.when(jnp.logical_and(sj >= 0, sj < nchunks))
        def _():
            drain_rows(gsems, b_s)
            start_scatter((bi + _NIDX - (_NBUF - 1)) % _NIDX, b_s)

    n_outer = (max(n0, n1) + 2 * _NBUF + _NIDX - 1) // _NIDX

    def group(t, carry):
        for bi in range(_NIDX):
            step(t * _NIDX + bi, bi)
        return carry

    lax.fori_loop(0, n_outer, group, 0)
    plsc.subcore_barrier()

    @pl.when(s < 15)
    def _():
        pltpu.sync_copy(acc_sh.at[pl.ds(s * _ACC_PER_TILE, _ACC_PER_TILE)],
                        out_hbm.at[c].at[pl.ds(s * _ACC_PER_TILE,
                                               _ACC_PER_TILE)])

    @pl.when(s == 15)
    def _():
        pltpu.sync_copy(acc_sh.at[pl.ds(15 * _ACC_PER_TILE,
                                        _ACC_ROWS - 15 * _ACC_PER_TILE)],
                        out_hbm.at[c].at[pl.ds(15 * _ACC_PER_TILE,
                                               _ACC_ROWS - 15 * _ACC_PER_TILE)])


def _make_sc_scatter(n0, n1):
    mesh = plsc.VectorSubcoreMesh(core_axis_name="c", subcore_axis_name="s")
    return pl.kernel(
        functools.partial(_sc_scatter_body, n0, n1),
        out_type=jax.ShapeDtypeStruct((2, _ACC_ROWS, _D), jnp.float32),
        mesh=mesh,
        scratch_types=[
            pltpu.VMEM((_NIDX, 2, _CH), jnp.int32),
            pltpu.VMEM((_NBUF, _CH, _D), jnp.float32),
            pltpu.VMEM_SHARED((_ACC_ROWS, _D), jnp.float32),
            pltpu.SemaphoreType.DMA((_NBUF,)),
            pltpu.SemaphoreType.DMA((_NBUF,)),
            pltpu.SemaphoreType.DMA((_NIDX,)),
        ])


def kernel(x, edge_index, in_W, in_b, l0_qparams, l0_lin_W, l0_lin_b, l0_qp_W,
           l0_qp_b, l1_qparams, l1_lin_W, l1_lin_b, l1_qp_W, l1_qp_b, out_W,
           out_b):
    sel = jnp.zeros((8, _D), jnp.float32).at[jnp.arange(4), jnp.arange(4)].set(
        1.0)

    def prep_layer(qparams, lin_W, lin_b, qp_W, qp_b):
        lWt = lin_W.T
        lb = (lin_b + qp_b).reshape(1, _D)
        qWt = jnp.zeros((8, _D), jnp.float32).at[0:4].set(qp_W.T)
        phi, th = qparams[:, 0], qparams[:, 1]
        c1 = jnp.zeros((8, _D), jnp.float32).at[0:4, 0].set(jnp.cos(th))
        c2 = jnp.zeros((8, _D), jnp.float32).at[0:4, 0].set(
            jnp.sin(th) * jnp.cos(phi))
        return lWt, lb, qWt, c1, c2, sel

    l0 = prep_layer(l0_qparams, l0_lin_W, l0_lin_b, l0_qp_W, l0_qp_b)
    l1 = prep_layer(l1_qparams, l1_lin_W, l1_lin_b, l1_qp_W, l1_qp_b)
    inWt = in_W.T
    inb = in_b.reshape(1, _D)
    outWt = out_W.T
    outb = out_b.reshape(1, _D)

    src = edge_index[0]
    dst = edge_index[1]
    n_edges = src.shape[0]
    n_pair = -(-n_edges // (16 * _CH))
    n0 = max(1, min(n_pair - 1, round(n_pair * _CORE0_FRAC)))
    n1 = n_pair - n0
    pad = 16 * n_pair * _CH - n_edges
    src_p = jnp.concatenate([src, jnp.zeros((pad,), jnp.int32)])
    dst_p = jnp.concatenate([dst, jnp.full((pad,), _N, jnp.int32)])
    ei = jnp.stack([src_p.reshape(-1, _CH), dst_p.reshape(-1, _CH)], axis=1)
    nmax = max(n0, n1)
    eia = ei[:16 * n0].reshape(16, n0, 2, _CH)
    eib1 = ei[16 * n0:].reshape(16, n1, 2, _CH)
    eia = jnp.pad(eia, ((0, 0), (0, nmax - n0), (0, 0), (0, 0)))
    eib1 = jnp.pad(eib1, ((0, 0), (0, nmax - n1), (0, 0), (0, 0)))
    eib = jnp.stack([eia, eib1], axis=1).reshape(_NW, nmax, 2, _CH)
    zblk = jnp.zeros((_ACC_PER_TILE, _D), jnp.float32)

    sc_scatter = _make_sc_scatter(n0, n1)

    h, xc = _tc_first(x, inWt, inb, *l0)
    acc = sc_scatter(xc, eib, zblk)
    h, xc = _tc_mid(h, xc, acc[0, :_N], acc[1, :_N], *l1)
    acc = sc_scatter(xc, eib, zblk)
    return _tc_last(h, xc, acc[0, :_N], acc[1, :_N], outWt, outb)


# speedup vs baseline: 13.3341x; 1.0299x over previous
import functools

import numpy as np
import jax
import jax.numpy as jnp
from jax import lax
from jax.experimental import pallas as pl
from jax.experimental.pallas import tpu as pltpu
from jax.experimental.pallas import tpu_sc as plsc

_N = 10000
_D = 128
_NW = 32
_CH = 88
_ACC_PER_TILE = 632
_ACC_ROWS = 10008
_ROW_BLK = 2000


def _xc_from_h(h, lWt, lb, qWt, c1, c2, sel):
    h8t = lax.dot_general(sel, h, (((1,), (1,)), ((), ())),
                          preferred_element_type=jnp.float32)
    a = jnp.tanh(h8t) * np.float32(np.pi / 2.0)
    z = jnp.cos(a) * c1[:, 0:1] - jnp.sin(a) * c2[:, 0:1]
    sub = lax.broadcasted_iota(jnp.int32, z.shape, 0)
    one = jnp.float32(1.0)
    t1 = z * jnp.where(sub == 0, one, pltpu.roll(z, 1, 0))
    t2 = t1 * jnp.where(sub < 2, one, pltpu.roll(t1, 2, 0))
    xq = lax.dot_general(t2, qWt, (((0,), (0,)), ((), ())),
                         preferred_element_type=jnp.float32)
    return jnp.dot(h, lWt, preferred_element_type=jnp.float32) + lb + xq


def _tc_first_body(x_ref, inWt_ref, inb_ref, lWt_ref, lb_ref, qWt_ref, c1_ref,
                   c2_ref, sel_ref, h_ref, xc_ref):
    h = jax.nn.relu(
        jnp.dot(x_ref[...], inWt_ref[...], preferred_element_type=jnp.float32)
        + inb_ref[...])
    h_ref[...] = h
    xc_ref[...] = _xc_from_h(h, lWt_ref[...], lb_ref[...], qWt_ref[...],
                             c1_ref[...], c2_ref[...], sel_ref[...])


def _tc_mid_body(hp_ref, xcp_ref, acc0_ref, acc1_ref, lWt_ref, lb_ref, qWt_ref,
                 c1_ref, c2_ref, sel_ref, h_ref, xc_ref):
    agg = acc0_ref[...] + acc1_ref[...] + xcp_ref[...]
    h = hp_ref[...] + jax.nn.relu(agg)
    h_ref[...] = h
    xc_ref[...] = _xc_from_h(h, lWt_ref[...], lb_ref[...], qWt_ref[...],
                             c1_ref[...], c2_ref[...], sel_ref[...])


def _tc_last_body(hp_ref, xcp_ref, acc0_ref, acc1_ref, outWt_ref, outb_ref,
                  out_ref):
    agg = acc0_ref[...] + acc1_ref[...] + xcp_ref[...]
    h = hp_ref[...] + jax.nn.relu(agg)
    out_ref[...] = (
        jnp.dot(h, outWt_ref[...], preferred_element_type=jnp.float32)
        + outb_ref[...])


_row_spec = pl.BlockSpec((_ROW_BLK, _D), lambda i: (i, 0))
_mat_spec = pl.BlockSpec((_D, _D), lambda i: (0, 0))
_vec_spec = pl.BlockSpec((1, _D), lambda i: (0, 0))
_qw_spec = pl.BlockSpec((8, _D), lambda i: (0, 0))
_GRID = (_N // _ROW_BLK,)

_hx_shape = (jax.ShapeDtypeStruct((_N, _D), jnp.float32),
             jax.ShapeDtypeStruct((_N, _D), jnp.float32))

_tc_first = pl.pallas_call(
    _tc_first_body, grid=_GRID,
    in_specs=[_row_spec, _mat_spec, _vec_spec, _mat_spec, _vec_spec, _qw_spec,
              _qw_spec, _qw_spec, _qw_spec],
    out_specs=(_row_spec, _row_spec),
    out_shape=_hx_shape)

_tc_mid = pl.pallas_call(
    _tc_mid_body, grid=_GRID,
    in_specs=[_row_spec, _row_spec, _row_spec, _row_spec, _mat_spec, _vec_spec,
              _qw_spec, _qw_spec, _qw_spec, _qw_spec],
    out_specs=(_row_spec, _row_spec),
    out_shape=_hx_shape)

_tc_last = pl.pallas_call(
    _tc_last_body, grid=_GRID,
    in_specs=[_row_spec, _row_spec, _row_spec, _row_spec, _mat_spec, _vec_spec],
    out_specs=_row_spec,
    out_shape=jax.ShapeDtypeStruct((_N, _D), jnp.float32))


_NBUF = 4
_NIDX = 8
_CORE0_FRAC = 0.70


def _sc_scatter_body(n0, n1, xc_hbm, eib_hbm, zblk_hbm, out_hbm,
                     idx_v, rows_v, acc_sh, gsems, ssems, isems):
    c = lax.axis_index("c")
    s = lax.axis_index("s")
    g = s * 2 + c
    nchunks = jnp.where(c == 0, n0, n1)

    @pl.when(s < 15)
    def _():
        pltpu.sync_copy(zblk_hbm,
                        acc_sh.at[pl.ds(s * _ACC_PER_TILE, _ACC_PER_TILE)])

    @pl.when(s == 15)
    def _():
        pltpu.sync_copy(zblk_hbm.at[pl.ds(0, _ACC_ROWS - 15 * _ACC_PER_TILE)],
                        acc_sh.at[pl.ds(15 * _ACC_PER_TILE,
                                        _ACC_ROWS - 15 * _ACC_PER_TILE)])

    plsc.subcore_barrier()

    def start_idx_load(j, bi):
        pltpu.async_copy(eib_hbm.at[g, j], idx_v.at[bi], isems.at[bi])

    def start_gather(bi, b):
        pltpu.async_copy(xc_hbm.at[idx_v.at[bi, 0]], rows_v.at[b],
                         gsems.at[b])

    def start_scatter(bi, b):
        pltpu.async_copy(rows_v.at[b], acc_sh.at[idx_v.at[bi, 1]], ssems.at[b],
                         add=True)

    def drain_rows(sems, b):
        pltpu.make_async_copy(xc_hbm.at[pl.ds(0, _CH)], rows_v.at[b],
                              sems.at[b]).wait()

    def drain_idx(bi):
        pltpu.make_async_copy(eib_hbm.at[0, 0], idx_v.at[bi],
                              isems.at[bi]).wait()

    for j0 in range(_NBUF):
        @pl.when(j0 < nchunks)
        def _():
            start_idx_load(j0, j0)

    def step(i, bi):
        b_g = bi % _NBUF
        b_s = (bi + 1) % _NBUF

        @pl.when(jnp.logical_and(i >= _NBUF, i < nchunks + _NBUF))
        def _():
            drain_rows(ssems, b_g)

        li = i + _NBUF

        @pl.when(li < nchunks)
        def _():
            start_idx_load(li, (bi + _NBUF) % _NIDX)

        @pl.when(i < nchunks)
        def _():
            drain_idx(bi)
            start_gather(bi, b_g)

        sj = i - (_NBUF - 1)

        @pl.when(jnp.logical_and(sj >= 0, sj < nchunks))
        def _():
            drain_rows(gsems, b_s)
            start_scatter((bi + _NIDX - (_NBUF - 1)) % _NIDX, b_s)

    n_outer = (max(n0, n1) + 2 * _NBUF + _NIDX - 1) // _NIDX

    def group(t, carry):
        for bi in range(_NIDX):
            step(t * _NIDX + bi, bi)
        return carry

    lax.fori_loop(0, n_outer, group, 0)
    plsc.subcore_barrier()

    @pl.when(s < 15)
    def _():
        pltpu.sync_copy(acc_sh.at[pl.ds(s * _ACC_PER_TILE, _ACC_PER_TILE)],
                        out_hbm.at[c].at[pl.ds(s * _ACC_PER_TILE,
                                               _ACC_PER_TILE)])

    @pl.when(s == 15)
    def _():
        pltpu.sync_copy(acc_sh.at[pl.ds(15 * _ACC_PER_TILE,
                                        _ACC_ROWS - 15 * _ACC_PER_TILE)],
                        out_hbm.at[c].at[pl.ds(15 * _ACC_PER_TILE,
                                               _ACC_ROWS - 15 * _ACC_PER_TILE)])


def _make_sc_scatter(n0, n1):
    mesh = plsc.VectorSubcoreMesh(core_axis_name="c", subcore_axis_name="s")
    return pl.kernel(
        functools.partial(_sc_scatter_body, n0, n1),
        out_type=jax.ShapeDtypeStruct((2, _ACC_ROWS, _D), jnp.float32),
        mesh=mesh,
        scratch_types=[
            pltpu.VMEM((_NIDX, 2, _CH), jnp.int32),
            pltpu.VMEM((_NBUF, _CH, _D), jnp.float32),
            pltpu.VMEM_SHARED((_ACC_ROWS, _D), jnp.float32),
            pltpu.SemaphoreType.DMA((_NBUF,)),
            pltpu.SemaphoreType.DMA((_NBUF,)),
            pltpu.SemaphoreType.DMA((_NIDX,)),
        ])


def kernel(x, edge_index, in_W, in_b, l0_qparams, l0_lin_W, l0_lin_b, l0_qp_W,
           l0_qp_b, l1_qparams, l1_lin_W, l1_lin_b, l1_qp_W, l1_qp_b, out_W,
           out_b):
    sel = jnp.zeros((8, _D), jnp.float32).at[jnp.arange(4), jnp.arange(4)].set(
        1.0)

    def prep_layer(qparams, lin_W, lin_b, qp_W, qp_b):
        lWt = lin_W.T
        lb = (lin_b + qp_b).reshape(1, _D)
        qWt = jnp.zeros((8, _D), jnp.float32).at[0:4].set(qp_W.T)
        phi, th = qparams[:, 0], qparams[:, 1]
        c1 = jnp.zeros((8, _D), jnp.float32).at[0:4, 0].set(jnp.cos(th))
        c2 = jnp.zeros((8, _D), jnp.float32).at[0:4, 0].set(
            jnp.sin(th) * jnp.cos(phi))
        return lWt, lb, qWt, c1, c2, sel

    l0 = prep_layer(l0_qparams, l0_lin_W, l0_lin_b, l0_qp_W, l0_qp_b)
    l1 = prep_layer(l1_qparams, l1_lin_W, l1_lin_b, l1_qp_W, l1_qp_b)
    inWt = in_W.T
    inb = in_b.reshape(1, _D)
    outWt = out_W.T
    outb = out_b.reshape(1, _D)

    src = edge_index[0]
    dst = edge_index[1]
    n_edges = src.shape[0]
    n_pair = -(-n_edges // (16 * _CH))
    n0 = max(1, min(n_pair - 1, round(n_pair * _CORE0_FRAC)))
    n1 = n_pair - n0
    pad = 16 * n_pair * _CH - n_edges
    src_p = jnp.concatenate([src, jnp.zeros((pad,), jnp.int32)])
    dst_p = jnp.concatenate([dst, jnp.full((pad,), _N, jnp.int32)])
    ei = jnp.stack([src_p.reshape(-1, _CH), dst_p.reshape(-1, _CH)], axis=1)
    nmax = max(n0, n1)
    eia = ei[:16 * n0].reshape(16, n0, 2, _CH)
    eib1 = ei[16 * n0:].reshape(16, n1, 2, _CH)
    eia = jnp.pad(eia, ((0, 0), (0, nmax - n0), (0, 0), (0, 0)))
    eib1 = jnp.pad(eib1, ((0, 0), (0, nmax - n1), (0, 0), (0, 0)))
    eib = jnp.stack([eia, eib1], axis=1).reshape(_NW, nmax, 2, _CH)
    zblk = jnp.zeros((_ACC_PER_TILE, _D), jnp.float32)

    sc_scatter = _make_sc_scatter(n0, n1)

    h, xc = _tc_first(x, inWt, inb, *l0)
    acc = sc_scatter(xc, eib, zblk)
    h, xc = _tc_mid(h, xc, acc[0], acc[1], *l1)
    acc = sc_scatter(xc, eib, zblk)
    return _tc_last(h, xc, acc[0], acc[1], outWt, outb)


# speedup vs baseline: 14.4219x; 1.0816x over previous
import functools

import numpy as np
import jax
import jax.numpy as jnp
from jax import lax
from jax.experimental import pallas as pl
from jax.experimental.pallas import tpu as pltpu
from jax.experimental.pallas import tpu_sc as plsc

_N = 10000
_D = 128
_NW = 32
_CH = 88
_ACC_PER_TILE = 632
_ACC_ROWS = 10008
_ROW_BLK = 2000


def _xc_from_h(h, lWt, lb, qWt, c1, c2, sel):
    h8t = lax.dot_general(sel, h, (((1,), (1,)), ((), ())),
                          preferred_element_type=jnp.float32)
    a = jnp.tanh(h8t) * np.float32(np.pi / 2.0)
    z = jnp.cos(a) * c1[:, 0:1] - jnp.sin(a) * c2[:, 0:1]
    sub = lax.broadcasted_iota(jnp.int32, z.shape, 0)
    one = jnp.float32(1.0)
    t1 = z * jnp.where(sub == 0, one, pltpu.roll(z, 1, 0))
    t2 = t1 * jnp.where(sub < 2, one, pltpu.roll(t1, 2, 0))
    xq = lax.dot_general(t2, qWt, (((0,), (0,)), ((), ())),
                         preferred_element_type=jnp.float32)
    return jnp.dot(h, lWt, preferred_element_type=jnp.float32) + lb + xq


def _tc_first_body(x_ref, inWt_ref, inb_ref, lWt_ref, lb_ref, qWt_ref, c1_ref,
                   c2_ref, sel_ref, h_ref, xc_ref):
    h = jax.nn.relu(
        jnp.dot(x_ref[...], inWt_ref[...], preferred_element_type=jnp.float32)
        + inb_ref[...])
    h_ref[...] = h
    xc_ref[...] = _xc_from_h(h, lWt_ref[...], lb_ref[...], qWt_ref[...],
                             c1_ref[...], c2_ref[...], sel_ref[...])


def _tc_mid_body(hp_ref, xcp_ref, acc0_ref, acc1_ref, lWt_ref, lb_ref, qWt_ref,
                 c1_ref, c2_ref, sel_ref, h_ref, xc_ref):
    agg = acc0_ref[...] + acc1_ref[...] + xcp_ref[...]
    h = hp_ref[...] + jax.nn.relu(agg)
    h_ref[...] = h
    xc_ref[...] = _xc_from_h(h, lWt_ref[...], lb_ref[...], qWt_ref[...],
                             c1_ref[...], c2_ref[...], sel_ref[...])


def _tc_last_body(hp_ref, xcp_ref, acc0_ref, acc1_ref, outWt_ref, outb_ref,
                  out_ref):
    agg = acc0_ref[...] + acc1_ref[...] + xcp_ref[...]
    h = hp_ref[...] + jax.nn.relu(agg)
    out_ref[...] = (
        jnp.dot(h, outWt_ref[...], preferred_element_type=jnp.float32)
        + outb_ref[...])


_row_spec = pl.BlockSpec((_ROW_BLK, _D), lambda i: (i, 0))
_mat_spec = pl.BlockSpec((_D, _D), lambda i: (0, 0))
_vec_spec = pl.BlockSpec((1, _D), lambda i: (0, 0))
_qw_spec = pl.BlockSpec((8, _D), lambda i: (0, 0))
_GRID = (_N // _ROW_BLK,)

_hx_shape = (jax.ShapeDtypeStruct((_N, _D), jnp.float32),
             jax.ShapeDtypeStruct((_N, _D), jnp.float32))

_tc_first = pl.pallas_call(
    _tc_first_body, grid=_GRID,
    in_specs=[_row_spec, _mat_spec, _vec_spec, _mat_spec, _vec_spec, _qw_spec,
              _qw_spec, _qw_spec, _qw_spec],
    out_specs=(_row_spec, _row_spec),
    out_shape=_hx_shape)

_tc_mid = pl.pallas_call(
    _tc_mid_body, grid=_GRID,
    in_specs=[_row_spec, _row_spec, _row_spec, _row_spec, _mat_spec, _vec_spec,
              _qw_spec, _qw_spec, _qw_spec, _qw_spec],
    out_specs=(_row_spec, _row_spec),
    out_shape=_hx_shape)

_tc_last = pl.pallas_call(
    _tc_last_body, grid=_GRID,
    in_specs=[_row_spec, _row_spec, _row_spec, _row_spec, _mat_spec, _vec_spec],
    out_specs=_row_spec,
    out_shape=jax.ShapeDtypeStruct((_N, _D), jnp.float32))


_NBUF = 4
_NIDX = 8
_CORE0_FRAC = 0.70


def _sc_scatter_body(n0, n1, xc_hbm, srcf_hbm, dstf_hbm, zblk_hbm, out0_hbm,
                     out1_hbm, idx_v, rows_v, acc_sh, gsems, ssems, isems):
    c = lax.axis_index("c")
    s = lax.axis_index("s")
    nchunks = jnp.where(c == 0, n0, n1)
    cbase = jnp.where(c == 0, s * n0, 16 * n0 + s * n1)

    @pl.when(s < 15)
    def _():
        pltpu.sync_copy(zblk_hbm,
                        acc_sh.at[pl.ds(s * _ACC_PER_TILE, _ACC_PER_TILE)])

    @pl.when(s == 15)
    def _():
        pltpu.sync_copy(zblk_hbm.at[pl.ds(0, _ACC_ROWS - 15 * _ACC_PER_TILE)],
                        acc_sh.at[pl.ds(15 * _ACC_PER_TILE,
                                        _ACC_ROWS - 15 * _ACC_PER_TILE)])

    plsc.subcore_barrier()

    def start_idx_load(j, bi):
        e0 = (cbase + j) * _CH
        pltpu.async_copy(srcf_hbm.at[pl.ds(e0, _CH)], idx_v.at[bi, 0],
                         isems.at[bi])
        pltpu.async_copy(dstf_hbm.at[pl.ds(e0, _CH)], idx_v.at[bi, 1],
                         isems.at[bi])

    def start_gather(bi, b):
        pltpu.async_copy(xc_hbm.at[idx_v.at[bi, 0]], rows_v.at[b],
                         gsems.at[b])

    def start_scatter(bi, b):
        pltpu.async_copy(rows_v.at[b], acc_sh.at[idx_v.at[bi, 1]], ssems.at[b],
                         add=True)

    def drain_rows(sems, b):
        pltpu.make_async_copy(xc_hbm.at[pl.ds(0, _CH)], rows_v.at[b],
                              sems.at[b]).wait()

    def drain_idx(bi):
        pltpu.make_async_copy(srcf_hbm.at[pl.ds(0, _CH)], idx_v.at[bi, 0],
                              isems.at[bi]).wait()
        pltpu.make_async_copy(dstf_hbm.at[pl.ds(0, _CH)], idx_v.at[bi, 1],
                              isems.at[bi]).wait()

    for j0 in range(_NBUF):
        @pl.when(j0 < nchunks)
        def _():
            start_idx_load(j0, j0)

    def step(i, bi):
        b_g = bi % _NBUF
        b_s = (bi + 1) % _NBUF

        @pl.when(jnp.logical_and(i >= _NBUF, i < nchunks + _NBUF))
        def _():
            drain_rows(ssems, b_g)

        li = i + _NBUF

        @pl.when(li < nchunks)
        def _():
            start_idx_load(li, (bi + _NBUF) % _NIDX)

        @pl.when(i < nchunks)
        def _():
            drain_idx(bi)
            start_gather(bi, b_g)

        sj = i - (_NBUF - 1)

        @pl.when(jnp.logical_and(sj >= 0, sj < nchunks))
        def _():
            drain_rows(gsems, b_s)
            start_scatter((bi + _NIDX - (_NBUF - 1)) % _NIDX, b_s)

    n_outer = (max(n0, n1) + 2 * _NBUF + _NIDX - 1) // _NIDX

    def group(t, carry):
        for bi in range(_NIDX):
            step(t * _NIDX + bi, bi)
        return carry

    lax.fori_loop(0, n_outer, group, 0)
    plsc.subcore_barrier()

    last = _ACC_ROWS - 15 * _ACC_PER_TILE
    nrows = jnp.where(s < 15, _ACC_PER_TILE, last)

    def write_out(out_hbm):
        @pl.when(s < 15)
        def _():
            pltpu.sync_copy(acc_sh.at[pl.ds(s * _ACC_PER_TILE, _ACC_PER_TILE)],
                            out_hbm.at[pl.ds(s * _ACC_PER_TILE,
                                             _ACC_PER_TILE)])

        @pl.when(s == 15)
        def _():
            pltpu.sync_copy(acc_sh.at[pl.ds(15 * _ACC_PER_TILE, last)],
                            out_hbm.at[pl.ds(15 * _ACC_PER_TILE, last)])

    @pl.when(c == 0)
    def _():
        write_out(out0_hbm)

    @pl.when(c == 1)
    def _():
        write_out(out1_hbm)


def _make_sc_scatter(n0, n1):
    mesh = plsc.VectorSubcoreMesh(core_axis_name="c", subcore_axis_name="s")
    return pl.kernel(
        functools.partial(_sc_scatter_body, n0, n1),
        out_type=(jax.ShapeDtypeStruct((_ACC_ROWS, _D), jnp.float32),
                  jax.ShapeDtypeStruct((_ACC_ROWS, _D), jnp.float32)),
        mesh=mesh,
        scratch_types=[
            pltpu.VMEM((_NIDX, 2, _CH), jnp.int32),
            pltpu.VMEM((_NBUF, _CH, _D), jnp.float32),
            pltpu.VMEM_SHARED((_ACC_ROWS, _D), jnp.float32),
            pltpu.SemaphoreType.DMA((_NBUF,)),
            pltpu.SemaphoreType.DMA((_NBUF,)),
            pltpu.SemaphoreType.DMA((_NIDX,)),
        ])


def kernel(x, edge_index, in_W, in_b, l0_qparams, l0_lin_W, l0_lin_b, l0_qp_W,
           l0_qp_b, l1_qparams, l1_lin_W, l1_lin_b, l1_qp_W, l1_qp_b, out_W,
           out_b):
    sel = jnp.zeros((8, _D), jnp.float32).at[jnp.arange(4), jnp.arange(4)].set(
        1.0)

    def prep_layer(qparams, lin_W, lin_b, qp_W, qp_b):
        lWt = lin_W.T
        lb = (lin_b + qp_b).reshape(1, _D)
        qWt = jnp.zeros((8, _D), jnp.float32).at[0:4].set(qp_W.T)
        phi, th = qparams[:, 0], qparams[:, 1]
        c1 = jnp.zeros((8, _D), jnp.float32).at[0:4, 0].set(jnp.cos(th))
        c2 = jnp.zeros((8, _D), jnp.float32).at[0:4, 0].set(
            jnp.sin(th) * jnp.cos(phi))
        return lWt, lb, qWt, c1, c2, sel

    l0 = prep_layer(l0_qparams, l0_lin_W, l0_lin_b, l0_qp_W, l0_qp_b)
    l1 = prep_layer(l1_qparams, l1_lin_W, l1_lin_b, l1_qp_W, l1_qp_b)
    inWt = in_W.T
    inb = in_b.reshape(1, _D)
    outWt = out_W.T
    outb = out_b.reshape(1, _D)

    n_edges = edge_index.shape[1]
    n_pair = -(-n_edges // (16 * _CH))
    n0 = max(1, min(n_pair - 1, round(n_pair * _CORE0_FRAC)))
    n1 = n_pair - n0
    pad = 16 * n_pair * _CH - n_edges
    srcf = jnp.concatenate([edge_index[0], jnp.zeros((pad,), jnp.int32)])
    dstf = jnp.concatenate([edge_index[1], jnp.full((pad,), _N, jnp.int32)])
    zblk = jnp.zeros((_ACC_PER_TILE, _D), jnp.float32)

    sc_scatter = _make_sc_scatter(n0, n1)

    h, xc = _tc_first(x, inWt, inb, *l0)
    acc0, acc1 = sc_scatter(xc, srcf, dstf, zblk)
    h, xc = _tc_mid(h, xc, acc0, acc1, *l1)
    acc0, acc1 = sc_scatter(xc, srcf, dstf, zblk)
    return _tc_last(h, xc, acc0, acc1, outWt, outb)


# speedup vs baseline: 15.1146x; 1.0480x over previous
import functools

import numpy as np
import jax
import jax.numpy as jnp
from jax import lax
from jax.experimental import pallas as pl
from jax.experimental.pallas import tpu as pltpu
from jax.experimental.pallas import tpu_sc as plsc

_N = 10000
_D = 128
_NW = 32
_CH = 72
_ACC_PER_TILE = 632
_ACC_ROWS = 10008
_ROW_BLK = 2000


def _xc_from_h(h, lWt, lb, qWt, c1, c2, sel):
    h8t = lax.dot_general(sel, h, (((1,), (1,)), ((), ())),
                          preferred_element_type=jnp.float32)
    a = jnp.tanh(h8t) * np.float32(np.pi / 2.0)
    z = jnp.cos(a) * c1[:, 0:1] - jnp.sin(a) * c2[:, 0:1]
    sub = lax.broadcasted_iota(jnp.int32, z.shape, 0)
    one = jnp.float32(1.0)
    t1 = z * jnp.where(sub == 0, one, pltpu.roll(z, 1, 0))
    t2 = t1 * jnp.where(sub < 2, one, pltpu.roll(t1, 2, 0))
    xq = lax.dot_general(t2, qWt, (((0,), (0,)), ((), ())),
                         preferred_element_type=jnp.float32)
    return jnp.dot(h, lWt, preferred_element_type=jnp.float32) + lb + xq


def _tc_first_body(x_ref, inWt_ref, inb_ref, lWt_ref, lb_ref, qWt_ref, c1_ref,
                   c2_ref, sel_ref, h_ref, xc_ref):
    h = jax.nn.relu(
        jnp.dot(x_ref[...], inWt_ref[...], preferred_element_type=jnp.float32)
        + inb_ref[...])
    h_ref[...] = h
    xc_ref[...] = _xc_from_h(h, lWt_ref[...], lb_ref[...], qWt_ref[...],
                             c1_ref[...], c2_ref[...], sel_ref[...])


def _tc_mid_body(hp_ref, xcp_ref, acc0_ref, acc1_ref, lWt_ref, lb_ref, qWt_ref,
                 c1_ref, c2_ref, sel_ref, h_ref, xc_ref):
    agg = acc0_ref[...] + acc1_ref[...] + xcp_ref[...]
    h = hp_ref[...] + jax.nn.relu(agg)
    h_ref[...] = h
    xc_ref[...] = _xc_from_h(h, lWt_ref[...], lb_ref[...], qWt_ref[...],
                             c1_ref[...], c2_ref[...], sel_ref[...])


def _tc_last_body(hp_ref, xcp_ref, acc0_ref, acc1_ref, outWt_ref, outb_ref,
                  out_ref):
    agg = acc0_ref[...] + acc1_ref[...] + xcp_ref[...]
    h = hp_ref[...] + jax.nn.relu(agg)
    out_ref[...] = (
        jnp.dot(h, outWt_ref[...], preferred_element_type=jnp.float32)
        + outb_ref[...])


_row_spec = pl.BlockSpec((_ROW_BLK, _D), lambda i: (i, 0))
_mat_spec = pl.BlockSpec((_D, _D), lambda i: (0, 0))
_vec_spec = pl.BlockSpec((1, _D), lambda i: (0, 0))
_qw_spec = pl.BlockSpec((8, _D), lambda i: (0, 0))
_GRID = (_N // _ROW_BLK,)

_hx_shape = (jax.ShapeDtypeStruct((_N, _D), jnp.float32),
             jax.ShapeDtypeStruct((_N, _D), jnp.float32))

_tc_first = pl.pallas_call(
    _tc_first_body, grid=_GRID,
    in_specs=[_row_spec, _mat_spec, _vec_spec, _mat_spec, _vec_spec, _qw_spec,
              _qw_spec, _qw_spec, _qw_spec],
    out_specs=(_row_spec, _row_spec),
    out_shape=_hx_shape)

_tc_mid = pl.pallas_call(
    _tc_mid_body, grid=_GRID,
    in_specs=[_row_spec, _row_spec, _row_spec, _row_spec, _mat_spec, _vec_spec,
              _qw_spec, _qw_spec, _qw_spec, _qw_spec],
    out_specs=(_row_spec, _row_spec),
    out_shape=_hx_shape)

_tc_last = pl.pallas_call(
    _tc_last_body, grid=_GRID,
    in_specs=[_row_spec, _row_spec, _row_spec, _row_spec, _mat_spec, _vec_spec],
    out_specs=_row_spec,
    out_shape=jax.ShapeDtypeStruct((_N, _D), jnp.float32))


_NBUF = 5
_NIDX = 10
_CORE0_FRAC = 0.714


def _sc_scatter_body(n0, n1, xc_hbm, srcf_hbm, dstf_hbm, zblk_hbm, out0_hbm,
                     out1_hbm, idx_v, rows_v, acc_sh, gsems, ssems, isems):
    c = lax.axis_index("c")
    s = lax.axis_index("s")
    nchunks = jnp.where(c == 0, n0, n1)
    cbase = jnp.where(c == 0, s * n0, 16 * n0 + s * n1)

    @pl.when(s < 15)
    def _():
        pltpu.sync_copy(zblk_hbm,
                        acc_sh.at[pl.ds(s * _ACC_PER_TILE, _ACC_PER_TILE)])

    @pl.when(s == 15)
    def _():
        pltpu.sync_copy(zblk_hbm.at[pl.ds(0, _ACC_ROWS - 15 * _ACC_PER_TILE)],
                        acc_sh.at[pl.ds(15 * _ACC_PER_TILE,
                                        _ACC_ROWS - 15 * _ACC_PER_TILE)])

    plsc.subcore_barrier()

    def start_idx_load(j, bi):
        e0 = (cbase + j) * _CH
        pltpu.async_copy(srcf_hbm.at[pl.ds(e0, _CH)], idx_v.at[bi, 0],
                         isems.at[bi])
        pltpu.async_copy(dstf_hbm.at[pl.ds(e0, _CH)], idx_v.at[bi, 1],
                         isems.at[bi])

    def start_gather(bi, b):
        pltpu.async_copy(xc_hbm.at[idx_v.at[bi, 0]], rows_v.at[b],
                         gsems.at[b])

    def start_scatter(bi, b):
        pltpu.async_copy(rows_v.at[b], acc_sh.at[idx_v.at[bi, 1]], ssems.at[b],
                         add=True)

    def drain_rows(sems, b):
        pltpu.make_async_copy(xc_hbm.at[pl.ds(0, _CH)], rows_v.at[b],
                              sems.at[b]).wait()

    def drain_idx(bi):
        pltpu.make_async_copy(srcf_hbm.at[pl.ds(0, _CH)], idx_v.at[bi, 0],
                              isems.at[bi]).wait()
        pltpu.make_async_copy(dstf_hbm.at[pl.ds(0, _CH)], idx_v.at[bi, 1],
                              isems.at[bi]).wait()

    for j0 in range(_NBUF):
        @pl.when(j0 < nchunks)
        def _():
            start_idx_load(j0, j0)

    def step(i, bi):
        b_g = bi % _NBUF
        b_s = (bi + 1) % _NBUF

        @pl.when(jnp.logical_and(i >= _NBUF, i < nchunks + _NBUF))
        def _():
            drain_rows(ssems, b_g)

        li = i + _NBUF

        @pl.when(li < nchunks)
        def _():
            start_idx_load(li, (bi + _NBUF) % _NIDX)

        @pl.when(i < nchunks)
        def _():
            drain_idx(bi)
            start_gather(bi, b_g)

        sj = i - (_NBUF - 1)

        @pl.when(jnp.logical_and(sj >= 0, sj < nchunks))
        def _():
            drain_rows(gsems, b_s)
            start_scatter((bi + _NIDX - (_NBUF - 1)) % _NIDX, b_s)

    n_outer = (max(n0, n1) + 2 * _NBUF + _NIDX - 1) // _NIDX

    def group(t, carry):
        for bi in range(_NIDX):
            step(t * _NIDX + bi, bi)
        return carry

    lax.fori_loop(0, n_outer, group, 0)
    plsc.subcore_barrier()

    last = _ACC_ROWS - 15 * _ACC_PER_TILE
    nrows = jnp.where(s < 15, _ACC_PER_TILE, last)

    def write_out(out_hbm):
        @pl.when(s < 15)
        def _():
            pltpu.sync_copy(acc_sh.at[pl.ds(s * _ACC_PER_TILE, _ACC_PER_TILE)],
                            out_hbm.at[pl.ds(s * _ACC_PER_TILE,
                                             _ACC_PER_TILE)])

        @pl.when(s == 15)
        def _():
            pltpu.sync_copy(acc_sh.at[pl.ds(15 * _ACC_PER_TILE, last)],
                            out_hbm.at[pl.ds(15 * _ACC_PER_TILE, last)])

    @pl.when(c == 0)
    def _():
        write_out(out0_hbm)

    @pl.when(c == 1)
    def _():
        write_out(out1_hbm)


def _make_sc_scatter(n0, n1):
    mesh = plsc.VectorSubcoreMesh(core_axis_name="c", subcore_axis_name="s")
    return pl.kernel(
        functools.partial(_sc_scatter_body, n0, n1),
        out_type=(jax.ShapeDtypeStruct((_ACC_ROWS, _D), jnp.float32),
                  jax.ShapeDtypeStruct((_ACC_ROWS, _D), jnp.float32)),
        mesh=mesh,
        scratch_types=[
            pltpu.VMEM((_NIDX, 2, _CH), jnp.int32),
            pltpu.VMEM((_NBUF, _CH, _D), jnp.float32),
            pltpu.VMEM_SHARED((_ACC_ROWS, _D), jnp.float32),
            pltpu.SemaphoreType.DMA((_NBUF,)),
            pltpu.SemaphoreType.DMA((_NBUF,)),
            pltpu.SemaphoreType.DMA((_NIDX,)),
        ])


def kernel(x, edge_index, in_W, in_b, l0_qparams, l0_lin_W, l0_lin_b, l0_qp_W,
           l0_qp_b, l1_qparams, l1_lin_W, l1_lin_b, l1_qp_W, l1_qp_b, out_W,
           out_b):
    sel = jnp.zeros((8, _D), jnp.float32).at[jnp.arange(4), jnp.arange(4)].set(
        1.0)

    def prep_layer(qparams, lin_W, lin_b, qp_W, qp_b):
        lWt = lin_W.T
        lb = (lin_b + qp_b).reshape(1, _D)
        qWt = jnp.zeros((8, _D), jnp.float32).at[0:4].set(qp_W.T)
        phi, th = qparams[:, 0], qparams[:, 1]
        c1 = jnp.zeros((8, _D), jnp.float32).at[0:4, 0].set(jnp.cos(th))
        c2 = jnp.zeros((8, _D), jnp.float32).at[0:4, 0].set(
            jnp.sin(th) * jnp.cos(phi))
        return lWt, lb, qWt, c1, c2, sel

    l0 = prep_layer(l0_qparams, l0_lin_W, l0_lin_b, l0_qp_W, l0_qp_b)
    l1 = prep_layer(l1_qparams, l1_lin_W, l1_lin_b, l1_qp_W, l1_qp_b)
    inWt = in_W.T
    inb = in_b.reshape(1, _D)
    outWt = out_W.T
    outb = out_b.reshape(1, _D)

    n_edges = edge_index.shape[1]
    n_pair = -(-n_edges // (16 * _CH))
    n0 = max(1, min(n_pair - 1, round(n_pair * _CORE0_FRAC)))
    n1 = n_pair - n0
    pad = 16 * n_pair * _CH - n_edges
    srcf = jnp.concatenate([edge_index[0], jnp.zeros((pad,), jnp.int32)])
    dstf = jnp.concatenate([edge_index[1], jnp.full((pad,), _N, jnp.int32)])
    zblk = jnp.zeros((_ACC_PER_TILE, _D), jnp.float32)

    sc_scatter = _make_sc_scatter(n0, n1)

    h, xc = _tc_first(x, inWt, inb, *l0)
    acc0, acc1 = sc_scatter(xc, srcf, dstf, zblk)
    h, xc = _tc_mid(h, xc, acc0, acc1, *l1)
    acc0, acc1 = sc_scatter(xc, srcf, dstf, zblk)
    return _tc_last(h, xc, acc0, acc1, outWt, outb)


# speedup vs baseline: 15.1932x; 1.0052x over previous
import functools

import numpy as np
import jax
import jax.numpy as jnp
from jax import lax
from jax.experimental import pallas as pl
from jax.experimental.pallas import tpu as pltpu
from jax.experimental.pallas import tpu_sc as plsc

_N = 10000
_D = 128
_CH = 72
_ACC_PER_TILE = 632
_ACC_ROWS = 10008
_ROW_BLK = 2000


def _xc_from_h(h, lWt, lb, qWt, c1, c2, sel):
    h8t = lax.dot_general(sel, h, (((1,), (1,)), ((), ())),
                          preferred_element_type=jnp.float32)
    a = jnp.tanh(h8t) * np.float32(np.pi / 2.0)
    z = jnp.cos(a) * c1[:, 0:1] - jnp.sin(a) * c2[:, 0:1]
    sub = lax.broadcasted_iota(jnp.int32, z.shape, 0)
    one = jnp.float32(1.0)
    t1 = z * jnp.where(sub == 0, one, pltpu.roll(z, 1, 0))
    t2 = t1 * jnp.where(sub < 2, one, pltpu.roll(t1, 2, 0))
    xq = lax.dot_general(t2, qWt, (((0,), (0,)), ((), ())),
                         preferred_element_type=jnp.float32)
    return jnp.dot(h, lWt, preferred_element_type=jnp.float32) + lb + xq


def _tc_first_body(x_ref, inWt_ref, inb_ref, lWt_ref, lb_ref, qWt_ref, c1_ref,
                   c2_ref, sel_ref, h_ref, xc_ref):
    h = jax.nn.relu(
        jnp.dot(x_ref[...], inWt_ref[...], preferred_element_type=jnp.float32)
        + inb_ref[...])
    h_ref[...] = h
    xc_ref[...] = _xc_from_h(h, lWt_ref[...], lb_ref[...], qWt_ref[...],
                             c1_ref[...], c2_ref[...], sel_ref[...])


def _tc_mid_body(hp_ref, xcp_ref, acc0_ref, acc1_ref, lWt_ref, lb_ref, qWt_ref,
                 c1_ref, c2_ref, sel_ref, h_ref, xc_ref):
    agg = acc0_ref[...] + acc1_ref[...] + xcp_ref[...]
    h = hp_ref[...] + jax.nn.relu(agg)
    h_ref[...] = h
    xc_ref[...] = _xc_from_h(h, lWt_ref[...], lb_ref[...], qWt_ref[...],
                             c1_ref[...], c2_ref[...], sel_ref[...])


def _tc_last_body(hp_ref, xcp_ref, acc0_ref, acc1_ref, outWt_ref, outb_ref,
                  out_ref):
    agg = acc0_ref[...] + acc1_ref[...] + xcp_ref[...]
    h = hp_ref[...] + jax.nn.relu(agg)
    out_ref[...] = (
        jnp.dot(h, outWt_ref[...], preferred_element_type=jnp.float32)
        + outb_ref[...])


_row_spec = pl.BlockSpec((_ROW_BLK, _D), lambda i: (i, 0))
_mat_spec = pl.BlockSpec((_D, _D), lambda i: (0, 0))
_vec_spec = pl.BlockSpec((1, _D), lambda i: (0, 0))
_qw_spec = pl.BlockSpec((8, _D), lambda i: (0, 0))
_GRID = (_N // _ROW_BLK,)

_hx_shape = (jax.ShapeDtypeStruct((_N, _D), jnp.float32),
             jax.ShapeDtypeStruct((_N, _D), jnp.float32))

_tc_first = pl.pallas_call(
    _tc_first_body, grid=_GRID,
    in_specs=[_row_spec, _mat_spec, _vec_spec, _mat_spec, _vec_spec, _qw_spec,
              _qw_spec, _qw_spec, _qw_spec],
    out_specs=(_row_spec, _row_spec),
    out_shape=_hx_shape)

_tc_mid = pl.pallas_call(
    _tc_mid_body, grid=_GRID,
    in_specs=[_row_spec, _row_spec, _row_spec, _row_spec, _mat_spec, _vec_spec,
              _qw_spec, _qw_spec, _qw_spec, _qw_spec],
    out_specs=(_row_spec, _row_spec),
    out_shape=_hx_shape)

_tc_last = pl.pallas_call(
    _tc_last_body, grid=_GRID,
    in_specs=[_row_spec, _row_spec, _row_spec, _row_spec, _mat_spec, _vec_spec],
    out_specs=_row_spec,
    out_shape=jax.ShapeDtypeStruct((_N, _D), jnp.float32))


_NBUF = 5
_NIDX = 10
_CORE0_FRAC = 0.714


def _sc_scatter_body(n0, n1, xc_hbm, srcf_hbm, dstf_hbm, zblk_hbm, out0_hbm,
                     out1_hbm, idx_v, rows_v, acc_sh, gsems, ssems, isems,
                     zsem):
    c = lax.axis_index("c")
    s = lax.axis_index("s")
    nchunks = jnp.where(c == 0, n0, n1)
    cbase = jnp.where(c == 0, s * n0, 16 * n0 + s * n1)

    zlast = _ACC_ROWS - 15 * _ACC_PER_TILE

    @pl.when(s < 15)
    def _():
        pltpu.async_copy(zblk_hbm,
                         acc_sh.at[pl.ds(s * _ACC_PER_TILE, _ACC_PER_TILE)],
                         zsem)

    @pl.when(s == 15)
    def _():
        pltpu.async_copy(zblk_hbm.at[pl.ds(0, zlast)],
                         acc_sh.at[pl.ds(15 * _ACC_PER_TILE, zlast)], zsem)

    def wait_zero():
        @pl.when(s < 15)
        def _():
            pltpu.make_async_copy(
                zblk_hbm,
                acc_sh.at[pl.ds(s * _ACC_PER_TILE, _ACC_PER_TILE)],
                zsem).wait()

        @pl.when(s == 15)
        def _():
            pltpu.make_async_copy(
                zblk_hbm.at[pl.ds(0, zlast)],
                acc_sh.at[pl.ds(15 * _ACC_PER_TILE, zlast)], zsem).wait()

    def start_idx_load(j, bi):
        e0 = (cbase + j) * _CH
        pltpu.async_copy(srcf_hbm.at[pl.ds(e0, _CH)], idx_v.at[bi, 0],
                         isems.at[bi])
        pltpu.async_copy(dstf_hbm.at[pl.ds(e0, _CH)], idx_v.at[bi, 1],
                         isems.at[bi])

    def start_gather(bi, b):
        pltpu.async_copy(xc_hbm.at[idx_v.at[bi, 0]], rows_v.at[b],
                         gsems.at[b])

    def start_scatter(bi, b):
        pltpu.async_copy(rows_v.at[b], acc_sh.at[idx_v.at[bi, 1]], ssems.at[b],
                         add=True)

    def drain_rows(sems, b):
        pltpu.make_async_copy(xc_hbm.at[pl.ds(0, _CH)], rows_v.at[b],
                              sems.at[b]).wait()

    def drain_idx(bi):
        pltpu.make_async_copy(srcf_hbm.at[pl.ds(0, _CH)], idx_v.at[bi, 0],
                              isems.at[bi]).wait()
        pltpu.make_async_copy(dstf_hbm.at[pl.ds(0, _CH)], idx_v.at[bi, 1],
                              isems.at[bi]).wait()

    for j0 in range(_NBUF):
        @pl.when(j0 < nchunks)
        def _():
            start_idx_load(j0, j0)

    def step(i, bi):
        b_g = bi % _NBUF
        b_s = (bi + 1) % _NBUF

        @pl.when(jnp.logical_and(i >= _NBUF, i < nchunks + _NBUF))
        def _():
            drain_rows(ssems, b_g)

        li = i + _NBUF

        @pl.when(li < nchunks)
        def _():
            start_idx_load(li, (bi + _NBUF) % _NIDX)

        @pl.when(i < nchunks)
        def _():
            drain_idx(bi)
            start_gather(bi, b_g)

        if bi == _NBUF - 1:
            @pl.when(i == _NBUF - 1)
            def _():
                wait_zero()
                plsc.subcore_barrier()

        sj = i - (_NBUF - 1)

        @pl.when(jnp.logical_and(sj >= 0, sj < nchunks))
        def _():
            drain_rows(gsems, b_s)
            start_scatter((bi + _NIDX - (_NBUF - 1)) % _NIDX, b_s)

    n_outer = (max(n0, n1) + 2 * _NBUF + _NIDX - 1) // _NIDX

    def group(t, carry):
        for bi in range(_NIDX):
            step(t * _NIDX + bi, bi)
        return carry

    lax.fori_loop(0, n_outer, group, 0)
    plsc.subcore_barrier()

    last = _ACC_ROWS - 15 * _ACC_PER_TILE

    def write_out(out_hbm):
        @pl.when(s < 15)
        def _():
            pltpu.sync_copy(acc_sh.at[pl.ds(s * _ACC_PER_TILE, _ACC_PER_TILE)],
                            out_hbm.at[pl.ds(s * _ACC_PER_TILE,
                                             _ACC_PER_TILE)])

        @pl.when(s == 15)
        def _():
            pltpu.sync_copy(acc_sh.at[pl.ds(15 * _ACC_PER_TILE, last)],
                            out_hbm.at[pl.ds(15 * _ACC_PER_TILE, last)])

    @pl.when(c == 0)
    def _():
        write_out(out0_hbm)

    @pl.when(c == 1)
    def _():
        write_out(out1_hbm)


def _make_sc_scatter(n0, n1):
    mesh = plsc.VectorSubcoreMesh(core_axis_name="c", subcore_axis_name="s")
    return pl.kernel(
        functools.partial(_sc_scatter_body, n0, n1),
        out_type=(jax.ShapeDtypeStruct((_ACC_ROWS, _D), jnp.float32),
                  jax.ShapeDtypeStruct((_ACC_ROWS, _D), jnp.float32)),
        mesh=mesh,
        scratch_types=[
            pltpu.VMEM((_NIDX, 2, _CH), jnp.int32),
            pltpu.VMEM((_NBUF, _CH, _D), jnp.float32),
            pltpu.VMEM_SHARED((_ACC_ROWS, _D), jnp.float32),
            pltpu.SemaphoreType.DMA((_NBUF,)),
            pltpu.SemaphoreType.DMA((_NBUF,)),
            pltpu.SemaphoreType.DMA((_NIDX,)),
            pltpu.SemaphoreType.DMA,
        ])


def kernel(x, edge_index, in_W, in_b, l0_qparams, l0_lin_W, l0_lin_b, l0_qp_W,
           l0_qp_b, l1_qparams, l1_lin_W, l1_lin_b, l1_qp_W, l1_qp_b, out_W,
           out_b):
    sel = jnp.zeros((8, _D), jnp.float32).at[jnp.arange(4), jnp.arange(4)].set(
        1.0)

    def prep_layer(qparams, lin_W, lin_b, qp_W, qp_b):
        lWt = lin_W.T
        lb = (lin_b + qp_b).reshape(1, _D)
        qWt = jnp.zeros((8, _D), jnp.float32).at[0:4].set(qp_W.T)
        phi, th = qparams[:, 0], qparams[:, 1]
        c1 = jnp.zeros((8, _D), jnp.float32).at[0:4, 0].set(jnp.cos(th))
        c2 = jnp.zeros((8, _D), jnp.float32).at[0:4, 0].set(
            jnp.sin(th) * jnp.cos(phi))
        return lWt, lb, qWt, c1, c2, sel

    l0 = prep_layer(l0_qparams, l0_lin_W, l0_lin_b, l0_qp_W, l0_qp_b)
    l1 = prep_layer(l1_qparams, l1_lin_W, l1_lin_b, l1_qp_W, l1_qp_b)
    inWt = in_W.T
    inb = in_b.reshape(1, _D)
    outWt = out_W.T
    outb = out_b.reshape(1, _D)

    n_edges = edge_index.shape[1]
    n_pair = -(-n_edges // (16 * _CH))
    n0 = max(1, min(n_pair - 1, round(n_pair * _CORE0_FRAC)))
    n1 = n_pair - n0
    pad = 16 * n_pair * _CH - n_edges
    srcf = jnp.concatenate([edge_index[0], jnp.zeros((pad,), jnp.int32)])
    dstf = jnp.concatenate([edge_index[1], jnp.full((pad,), _N, jnp.int32)])
    zblk = jnp.zeros((_ACC_PER_TILE, _D), jnp.float32)

    sc_scatter = _make_sc_scatter(n0, n1)

    h, xc = _tc_first(x, inWt, inb, *l0)
    acc0, acc1 = sc_scatter(xc, srcf, dstf, zblk)
    h, xc = _tc_mid(h, xc, acc0, acc1, *l1)
    acc0, acc1 = sc_scatter(xc, srcf, dstf, zblk)
    return _tc_last(h, xc, acc0, acc1, outWt, outb)
